# baseline (speedup 1.0000x reference)
import functools

import jax
import jax.numpy as jnp
from jax import lax
from jax.experimental import pallas as pl
from jax.experimental.pallas import tpu as pltpu
from jax.experimental.pallas import tpu_sc as plsc

_B = 4096
_L = 200
_EMB = 64
_HID = 128
_NCLS = 100
_V = 1000000

_CA = 96
_CB = 104

_NC = 2
_NS = 16
_NW = _NC * _NS
_BPW = _B // _NW

_CW = 256
_NFULL = 999936 // _CW
_PERW = _NFULL // _NW
_NEXTRA = _NFULL - _PERW * _NW
_TAIL = _V - _NFULL * _CW

_mesh = plsc.VectorSubcoreMesh(core_axis_name="c", subcore_axis_name="s")

_slab = jax.ShapeDtypeStruct((_V * 16,), jnp.float32)


@functools.partial(
    pl.kernel,
    out_type=(_slab, _slab, _slab, _slab),
    mesh=_mesh,
    scratch_types=[
        pltpu.VMEM((2, 64, _CW), jnp.float32),
        pltpu.VMEM((2 * 16 * _CW,), jnp.float32),
        pltpu.VMEM((2 * 16 * _CW,), jnp.float32),
        pltpu.VMEM((2 * 16 * _CW,), jnp.float32),
        pltpu.VMEM((2 * 16 * _CW,), jnp.float32),
        pltpu.VMEM((64 * 64,), jnp.float32),
        pltpu.SemaphoreType.DMA,
        pltpu.SemaphoreType.DMA,
    ],
    compiler_params=pltpu.CompilerParams(
        use_tc_tiling_on_sc=True, needs_layout_passes=False),
)
def _detile(embt, tail_lin, x0, x1, x2, x3, chunk, ob0, ob1, ob2, ob3,
            tailbuf, in_sem, out_sem):
    wid = lax.axis_index("s") * _NC + lax.axis_index("c")
    start = wid * _PERW
    xs = (x0, x1, x2, x3)
    obs = (ob0, ob1, ob2, ob3)
    _CWW = 16 * _CW
    iota = lax.iota(jnp.int32, 16)
    rows16 = [iota + 16 * k for k in range(4)]
    perms = [(iota + s) % 16 for s in range(16)]
    wvecs = [16 * p + iota for p in perms]

    def in_copies(c, bank):
        return [
            pltpu.make_async_copy(
                embt.at[pl.ds(8 * g, 8), pl.ds(c * _CW, _CW)],
                chunk.at[bank, pl.ds(8 * g, 8)], in_sem)
            for g in range(8)
        ]

    def out_copy(c, bank, k):
        return pltpu.make_async_copy(
            obs[k].at[pl.ds(bank * _CWW, _CWW)],
            xs[k].at[pl.ds(c * _CWW, _CWW)], out_sem)

    def do_chunk(bank, ncols):
        def jbody(jb, _):
            j0 = jb * 16
            for k in range(4):
                for s in range(16):
                    cols = j0 + perms[s]
                    v = plsc.load_gather(chunk.at[bank], [rows16[k], cols])
                    plsc.store_scatter(
                        obs[k], [bank * _CWW + j0 * 16 + wvecs[s]], v)
            return 0
        lax.fori_loop(0, ncols // 16, jbody, 0)

    for cp in in_copies(start, 0):
        cp.start()
    for cp in in_copies(start + 1, 1):
        cp.start()

    def outer(i2, _):
        for bank in range(2):
            idx = i2 * 2 + bank
            c = start + idx
            for cp in in_copies(c, bank):
                cp.wait()

            @pl.when(idx >= 2)
            def _():
                for k in range(4):
                    out_copy(c - 2, bank, k).wait()

            do_chunk(bank, _CW)
            for k in range(4):
                out_copy(c, bank, k).start()

            @pl.when(idx + 2 < _PERW)
            def _():
                for cp in in_copies(c + 2, bank):
                    cp.start()

        return 0

    lax.fori_loop(0, _PERW // 2, outer, 0)
    for bank in range(2):
        for k in range(4):
            out_copy(start + _PERW - 2 + bank, bank, k).wait()

    @pl.when(wid < _NEXTRA)
    def _():
        c = _NW * _PERW + wid
        for cp in in_copies(c, 0):
            cp.start()
        for cp in in_copies(c, 0):
            cp.wait()
        do_chunk(0, _CW)
        for k in range(4):
            out_copy(c, 0, k).start()
        for k in range(4):
            out_copy(c, 0, k).wait()

    @pl.when(wid == _NW - 1)
    def _():
        tin = pltpu.make_async_copy(tail_lin, tailbuf, in_sem)
        tin.start()
        tin.wait()

        def trow(r, _):
            for k in range(4):
                obs[k][pl.ds(r * 16, 16)] = tailbuf[pl.ds(r * 64 + k * 16, 16)]
            return 0

        lax.fori_loop(0, _TAIL, trow, 0)
        for k in range(4):
            t = pltpu.make_async_copy(
                obs[k].at[pl.ds(0, _TAIL * 16)],
                xs[k].at[pl.ds(_NFULL * _CWW, _TAIL * 16)], out_sem)
            t.start()
            t.wait()


@functools.partial(
    pl.kernel,
    out_type=jax.ShapeDtypeStruct((_B, _EMB), jnp.float32),
    mesh=_mesh,
    scratch_types=[
        pltpu.VMEM((_BPW, _CA), jnp.int32),
        pltpu.VMEM((_BPW, _CB), jnp.int32),
        pltpu.VMEM((4, 4, _CA, 16), jnp.float32),
        pltpu.VMEM((4, 4, _CB, 16), jnp.float32),
        pltpu.VMEM((_BPW, _EMB), jnp.float32),
        pltpu.SemaphoreType.DMA,
        pltpu.SemaphoreType.DMA,
    ],
    compiler_params=pltpu.CompilerParams(use_tc_tiling_on_sc=False),
)
def _pooled(texta, textb, t0, t1, t2, t3, out, idxa, idxb, rowsa, rowsb,
            acc, sema, semb):
    wid = lax.axis_index("s") * _NC + lax.axis_index("c")
    base = wid * _BPW
    tabs = (t0, t1, t2, t3)

    pltpu.sync_copy(texta.at[pl.ds(base, _BPW)], idxa)
    pltpu.sync_copy(textb.at[pl.ds(base, _BPW)], idxb)

    def copies(row, bank):
        cs = []
        for k in range(4):
            cs.append(pltpu.make_async_copy(
                tabs[k].at[idxa.at[row]], rowsa.at[k, bank], sema))
            cs.append(pltpu.make_async_copy(
                tabs[k].at[idxb.at[row]], rowsb.at[k, bank], semb))
        return cs

    def accum(ref, bank, n, accs):
        def body(j, accs):
            out = list(accs)
            for u in range(8):
                r = j * 8 + u
                for k in range(4):
                    out[k] = out[k] + ref[k, bank, r]
            return tuple(out)
        return lax.fori_loop(0, n // 8, body, accs)

    for bank in range(4):
        for c in copies(bank, bank):
            c.start()

    zero = jnp.zeros((16,), jnp.float32)

    def outer(i, carry):
        for bank in range(4):
            row = i * 4 + bank
            for c in copies(row, bank):
                c.wait()
            accs = (zero, zero, zero, zero)
            accs = accum(rowsa, bank, _CA, accs)
            accs = accum(rowsb, bank, _CB, accs)
            for k in range(4):
                acc[row, pl.ds(k * 16, 16)] = accs[k]

            @pl.when(row + 4 < _BPW)
            def _():
                for c in copies(row + 4, bank):
                    c.start()

        return carry

    lax.fori_loop(0, _BPW // 4, outer, 0)

    pltpu.sync_copy(acc, out.at[pl.ds(base, _BPW)])


def _mlp_body(e_ref, w1_ref, b1_ref, w2_ref, b2_ref, o_ref):
    e = e_ref[...] * (1.0 / _L)
    h = jnp.tanh(
        lax.dot_general(e, w1_ref[...], (((1,), (0,)), ((), ())),
                        preferred_element_type=jnp.float32)
        + b1_ref[...])
    o_ref[...] = (
        lax.dot_general(h, w2_ref[...], (((1,), (0,)), ((), ())),
                        preferred_element_type=jnp.float32)
        + b2_ref[...])


_BB = 512


def _mlp(pooled, w1, b1, w2, b2):
    return pl.pallas_call(
        _mlp_body,
        grid=(_B // _BB,),
        in_specs=[
            pl.BlockSpec((_BB, _EMB), lambda i: (i, 0)),
            pl.BlockSpec((_EMB, _HID), lambda i: (0, 0)),
            pl.BlockSpec((1, _HID), lambda i: (0, 0)),
            pl.BlockSpec((_HID, _NCLS), lambda i: (0, 0)),
            pl.BlockSpec((1, _NCLS), lambda i: (0, 0)),
        ],
        out_specs=pl.BlockSpec((_BB, _NCLS), lambda i: (i, 0)),
        out_shape=jax.ShapeDtypeStruct((_B, _NCLS), jnp.float32),
    )(pooled, w1, b1.reshape(1, _HID), w2, b2.reshape(1, _NCLS))


def kernel(text, embed, w1, b1, w2, b2):
    embt = jnp.swapaxes(embed, 0, 1)
    tail_lin = embed[_NFULL * _CW:].reshape(-1)
    slabs = _detile(embt, tail_lin)
    tabs = [s.reshape(_V, 16) for s in slabs]
    texta = text[:, :_CA]
    textb = text[:, _CA:]
    pooled = _pooled(texta, textb, *tabs)
    return _mlp(pooled, w1, b1, w2, b2)


# speedup vs baseline: 1.0045x; 1.0045x over previous
import functools

import jax
import jax.numpy as jnp
from jax import lax
from jax.experimental import pallas as pl
from jax.experimental.pallas import tpu as pltpu
from jax.experimental.pallas import tpu_sc as plsc

_B = 4096
_L = 200
_EMB = 64
_HID = 128
_NCLS = 100
_V = 1000000

_CA = 96
_CB = 104

_NC = 2
_NS = 16
_NW = _NC * _NS
_BPW = _B // _NW

_CW = 256
_NFULL = 999936 // _CW
_PERW = _NFULL // _NW
_NEXTRA = _NFULL - _PERW * _NW
_TAIL = _V - _NFULL * _CW

_mesh = plsc.VectorSubcoreMesh(core_axis_name="c", subcore_axis_name="s")

_slab = jax.ShapeDtypeStruct((_V * 16,), jnp.float32)


@functools.partial(
    pl.kernel,
    out_type=(_slab, _slab, _slab, _slab),
    mesh=_mesh,
    scratch_types=[
        pltpu.VMEM((2, 64, _CW), jnp.float32),
        pltpu.VMEM((2 * 16 * _CW,), jnp.float32),
        pltpu.VMEM((2 * 16 * _CW,), jnp.float32),
        pltpu.VMEM((2 * 16 * _CW,), jnp.float32),
        pltpu.VMEM((2 * 16 * _CW,), jnp.float32),
        pltpu.VMEM((64 * 64,), jnp.float32),
        pltpu.SemaphoreType.DMA,
        pltpu.SemaphoreType.DMA,
    ],
    compiler_params=pltpu.CompilerParams(
        use_tc_tiling_on_sc=True, needs_layout_passes=False),
)
def _detile(embt, tail_lin, x0, x1, x2, x3, chunk, ob0, ob1, ob2, ob3,
            tailbuf, in_sem, out_sem):
    wid = lax.axis_index("s") * _NC + lax.axis_index("c")
    start = wid * _PERW
    xs = (x0, x1, x2, x3)
    obs = (ob0, ob1, ob2, ob3)
    _CWW = 16 * _CW
    iota = lax.iota(jnp.int32, 16)
    rows16 = [iota + 16 * k for k in range(4)]
    perms = [(iota + s) % 16 for s in range(16)]
    wvecs = [16 * p + iota for p in perms]

    def in_copies(c, bank):
        return [
            pltpu.make_async_copy(
                embt.at[pl.ds(8 * g, 8), pl.ds(c * _CW, _CW)],
                chunk.at[bank, pl.ds(8 * g, 8)], in_sem)
            for g in range(8)
        ]

    def out_copy(c, bank, k):
        return pltpu.make_async_copy(
            obs[k].at[pl.ds(bank * _CWW, _CWW)],
            xs[k].at[pl.ds(c * _CWW, _CWW)], out_sem)

    def do_chunk(bank, ncols):
        def jbody(jb, _):
            j0 = jb * 16
            for k in range(4):
                for s in range(16):
                    cols = j0 + perms[s]
                    v = plsc.load_gather(chunk.at[bank], [rows16[k], cols])
                    plsc.store_scatter(
                        obs[k], [bank * _CWW + j0 * 16 + wvecs[s]], v)
            return 0
        lax.fori_loop(0, ncols // 16, jbody, 0)

    for cp in in_copies(start, 0):
        cp.start()
    for cp in in_copies(start + 1, 1):
        cp.start()

    def outer(i2, _):
        for bank in range(2):
            idx = i2 * 2 + bank
            c = start + idx
            for cp in in_copies(c, bank):
                cp.wait()

            @pl.when(idx >= 2)
            def _():
                for k in range(4):
                    out_copy(c - 2, bank, k).wait()

            do_chunk(bank, _CW)
            for k in range(4):
                out_copy(c, bank, k).start()

            @pl.when(idx + 2 < _PERW)
            def _():
                for cp in in_copies(c + 2, bank):
                    cp.start()

        return 0

    lax.fori_loop(0, _PERW // 2, outer, 0)
    for bank in range(2):
        for k in range(4):
            out_copy(start + _PERW - 2 + bank, bank, k).wait()

    @pl.when(wid < _NEXTRA)
    def _():
        c = _NW * _PERW + wid
        for cp in in_copies(c, 0):
            cp.start()
        for cp in in_copies(c, 0):
            cp.wait()
        do_chunk(0, _CW)
        for k in range(4):
            out_copy(c, 0, k).start()
        for k in range(4):
            out_copy(c, 0, k).wait()

    @pl.when(wid == _NW - 1)
    def _():
        tin = pltpu.make_async_copy(tail_lin, tailbuf, in_sem)
        tin.start()
        tin.wait()

        def trow(r, _):
            for k in range(4):
                obs[k][pl.ds(r * 16, 16)] = tailbuf[pl.ds(r * 64 + k * 16, 16)]
            return 0

        lax.fori_loop(0, _TAIL, trow, 0)
        for k in range(4):
            t = pltpu.make_async_copy(
                obs[k].at[pl.ds(0, _TAIL * 16)],
                xs[k].at[pl.ds(_NFULL * _CWW, _TAIL * 16)], out_sem)
            t.start()
            t.wait()


@functools.partial(
    pl.kernel,
    out_type=jax.ShapeDtypeStruct((_B, _EMB), jnp.float32),
    mesh=_mesh,
    scratch_types=[
        pltpu.VMEM((_BPW, _CA), jnp.int32),
        pltpu.VMEM((_BPW, _CB), jnp.int32),
        pltpu.VMEM((4, 4, _CA, 16), jnp.float32),
        pltpu.VMEM((4, 4, _CB, 16), jnp.float32),
        pltpu.VMEM((_BPW, _EMB), jnp.float32),
        pltpu.SemaphoreType.DMA,
        pltpu.SemaphoreType.DMA,
    ],
    compiler_params=pltpu.CompilerParams(use_tc_tiling_on_sc=False),
)
def _pooled(texta, textb, t0, t1, t2, t3, out, idxa, idxb, rowsa, rowsb,
            acc, sema, semb):
    wid = lax.axis_index("s") * _NC + lax.axis_index("c")
    base = wid * _BPW
    tabs = (t0, t1, t2, t3)

    pltpu.sync_copy(texta.at[pl.ds(base, _BPW)], idxa)
    pltpu.sync_copy(textb.at[pl.ds(base, _BPW)], idxb)

    def copies(row, bank):
        cs = []
        for k in range(4):
            cs.append(pltpu.make_async_copy(
                tabs[k].at[idxa.at[row]], rowsa.at[k, bank], sema))
            cs.append(pltpu.make_async_copy(
                tabs[k].at[idxb.at[row]], rowsb.at[k, bank], semb))
        return cs

    def accum(ref, bank, n, accs):
        def body(j, accs):
            out = list(accs)
            for u in range(4):
                r = j * 4 + u
                for k in range(4):
                    out[k] = out[k] + ref[k, bank, r]
            return tuple(out)
        return lax.fori_loop(0, n // 4, body, accs)

    for bank in range(4):
        for c in copies(bank, bank):
            c.start()

    zero = jnp.zeros((16,), jnp.float32)

    def outer(i, carry):
        for bank in range(4):
            row = i * 4 + bank
            for c in copies(row, bank):
                c.wait()
            accs = (zero, zero, zero, zero)
            accs = accum(rowsa, bank, _CA, accs)
            accs = accum(rowsb, bank, _CB, accs)
            for k in range(4):
                acc[row, pl.ds(k * 16, 16)] = accs[k]

            @pl.when(row + 4 < _BPW)
            def _():
                for c in copies(row + 4, bank):
                    c.start()

        return carry

    lax.fori_loop(0, _BPW // 4, outer, 0)

    pltpu.sync_copy(acc, out.at[pl.ds(base, _BPW)])


def _mlp_body(e_ref, w1_ref, b1_ref, w2_ref, b2_ref, o_ref):
    e = e_ref[...] * (1.0 / _L)
    h = jnp.tanh(
        lax.dot_general(e, w1_ref[...], (((1,), (0,)), ((), ())),
                        preferred_element_type=jnp.float32)
        + b1_ref[...])
    o_ref[...] = (
        lax.dot_general(h, w2_ref[...], (((1,), (0,)), ((), ())),
                        preferred_element_type=jnp.float32)
        + b2_ref[...])


_BB = 512


def _mlp(pooled, w1, b1, w2, b2):
    return pl.pallas_call(
        _mlp_body,
        grid=(_B // _BB,),
        in_specs=[
            pl.BlockSpec((_BB, _EMB), lambda i: (i, 0)),
            pl.BlockSpec((_EMB, _HID), lambda i: (0, 0)),
            pl.BlockSpec((1, _HID), lambda i: (0, 0)),
            pl.BlockSpec((_HID, _NCLS), lambda i: (0, 0)),
            pl.BlockSpec((1, _NCLS), lambda i: (0, 0)),
        ],
        out_specs=pl.BlockSpec((_BB, _NCLS), lambda i: (i, 0)),
        out_shape=jax.ShapeDtypeStruct((_B, _NCLS), jnp.float32),
    )(pooled, w1, b1.reshape(1, _HID), w2, b2.reshape(1, _NCLS))


def kernel(text, embed, w1, b1, w2, b2):
    embt = jnp.swapaxes(embed, 0, 1)
    tail_lin = embed[_NFULL * _CW:].reshape(-1)
    slabs = _detile(embt, tail_lin)
    tabs = [s.reshape(_V, 16) for s in slabs]
    texta = text[:, :_CA]
    textb = text[:, _CA:]
    pooled = _pooled(texta, textb, *tabs)
    return _mlp(pooled, w1, b1, w2, b2)
